# baseline (speedup 1.0000x reference)
import functools

import jax
import jax.numpy as jnp
from jax import lax
from jax.experimental import pallas as pl
from jax.experimental.pallas import tpu as pltpu
from jax.experimental.pallas import tpu_sc as plsc

NOBS = 16384
CH = 10
NCOLS = 33
L = 16

_info = plsc.get_sparse_core_info()
NC = _info.num_cores
NS = _info.num_subcores
NW = NC * NS
BPW = NOBS // NW
RCH = 64
NWAVE = BPW // RCH

_mesh = plsc.VectorSubcoreMesh(core_axis_name="c", subcore_axis_name="s")

_f32 = jnp.float32
_i32 = jnp.int32


@functools.partial(
    pl.kernel,
    mesh=_mesh,
    compiler_params=pltpu.CompilerParams(
        needs_layout_passes=False, use_tc_tiling_on_sc=True),
    out_type=jax.ShapeDtypeStruct((NOBS,), _f32),
    scratch_types=[
        pltpu.VMEM((BPW,), _i32),
        pltpu.VMEM((BPW,), _i32),
        pltpu.VMEM((BPW,), _f32),
        [pltpu.VMEM((RCH, NCOLS), _f32) for _ in range(2)],
        pltpu.VMEM((RCH, 2), _i32),
        [pltpu.SemaphoreType.DMA for _ in range(2)],
        pltpu.SemaphoreType.DMA,
    ],
)
def _seaice_sc(geo_hbm, tab_hbm, s_hbm,
               row_v, col_v, s_v, rows_b, geo_i, sem_rows, sem_s):
    wid = lax.axis_index("s") * NC + lax.axis_index("c")
    base = wid * BPW

    lane0 = lax.iota(_i32, L)
    zero16 = lane0 - lane0
    one16 = zero16 + 1

    for g in range(NWAVE):
        pltpu.sync_copy(
            geo_hbm.at[pl.ds(base + g * RCH, RCH), :], geo_i)

        def geo_body(j, carry, g=g):
            i16 = j * L + lane0
            r = plsc.load_gather(geo_i, [i16, zero16])
            c = plsc.load_gather(geo_i, [i16, one16])
            off = g * RCH + j * L
            row_v[pl.ds(off, L)] = r
            col_v[pl.ds(off, L)] = c
            return carry

        lax.fori_loop(0, RCH // L, geo_body, 0)

    def fire_wave(w):
        b = w % 2
        buf = rows_b[b]

        def body(j, carry):
            rv = row_v[pl.ds(w * RCH + j * L, L)]
            for k in range(L):
                pltpu.async_copy(tab_hbm.at[pl.ds(rv[k], 1), :],
                                 buf.at[pl.ds(j * L + k, 1), :],
                                 sem_rows[b])
            return carry

        lax.fori_loop(0, RCH // L, body, 0)

    def drain_wave(w):
        b = w % 2
        pltpu.make_async_copy(
            tab_hbm.at[pl.ds(0, RCH), :], rows_b[b], sem_rows[b]).wait()

    lane = lax.iota(_i32, L)
    a0 = _f32(0.2)
    a1 = _f32(0.3)
    a2 = _f32(0.5)

    fire_wave(0)

    for w in range(NWAVE):
        if w + 1 < NWAVE:
            fire_wave(w + 1)

        drain_wave(w)
        buf = rows_b[w % 2]

        def s_body(j, carry, w=w, buf=buf):
            i16 = j * L + lane
            off = w * RCH + j * L
            c = col_v[pl.ds(off, L)]
            g0 = plsc.load_gather(buf, [i16, c])
            g1 = plsc.load_gather(buf, [i16, c + 1])
            g2 = plsc.load_gather(buf, [i16, c + 2])
            s_v[pl.ds(off, L)] = a0 * g0 + a1 * g1 + a2 * g2
            return carry

        lax.fori_loop(0, RCH // L, s_body, 0)

    pltpu.async_copy(s_v, s_hbm.at[pl.ds(base, BPW)], sem_s).wait()


_BLK = 2048


def _blend_body(s_ref, eo_ref, es_ref, o_ref):
    sv = s_ref[...]
    eo = eo_ref[...]
    es = es_ref[...]
    o_ref[...] = eo + sv * (es - eo)


def _blend_tc(s2, eo, es):
    return pl.pallas_call(
        _blend_body,
        grid=(NOBS // _BLK,),
        in_specs=[
            pl.BlockSpec((_BLK, 1), lambda i: (i, 0)),
            pl.BlockSpec((_BLK, CH), lambda i: (i, 0)),
            pl.BlockSpec((_BLK, CH), lambda i: (i, 0)),
        ],
        out_specs=pl.BlockSpec((_BLK, CH), lambda i: (i, 0)),
        out_shape=jax.ShapeDtypeStruct((NOBS, CH), _f32),
    )(s2, eo, es)


def kernel(geolocation, emis_ocean, emis_seaice, tsfc, seaice, seaice_background):
    del tsfc, seaice_background
    s = _seaice_sc(geolocation, seaice)
    out = _blend_tc(s.reshape(NOBS, 1), emis_ocean, emis_seaice)
    return (out, s)
